# baseline (speedup 1.0000x reference)
import functools
import jax
import jax.numpy as jnp
from jax import lax
from jax.experimental import pallas as pl
from jax.experimental.pallas import tpu as pltpu
from jax.experimental.pallas import tpu_sc as plsc

_E = 64
_N = 32768
_NC = 2
_NS = 16
_NW = _NC * _NS
_TPW = _N // _NW
_L = 16
_GROUPS = _TPW // _L


def _sc_body(x_hbm, mask_hbm, cnt_hbm, psum_hbm, xv, mv, cntv, psv):
    wid = lax.axis_index("s") * _NC + lax.axis_index("c")
    base = wid * _TPW
    pltpu.sync_copy(x_hbm.at[pl.ds(base * _E, _TPW * _E)], xv)
    pltpu.sync_copy(mask_hbm.at[pl.ds(base, _TPW)], mv)

    zeros = jnp.zeros((_L,), jnp.float32)
    for k in range(_E // _L):
        cntv[pl.ds(_L * k, _L)] = zeros
        psv[pl.ds(_L * k, _L)] = zeros

    lanes = lax.iota(jnp.int32, _L)

    @pl.loop(0, _GROUPS)
    def _grp(g):
        idx = g * (_L * _E) + lanes * _E
        valid = mv[pl.ds(g * _L, _L)] == 1
        best = plsc.load_gather(xv, [idx])
        bidx = jnp.zeros((_L,), jnp.int32)
        for e in range(1, _E):
            idx = idx + 1
            v = plsc.load_gather(xv, [idx])
            gt = v > best
            best = jnp.where(gt, v, best)
            bidx = jnp.where(gt, e, bidx)
        plsc.addupdate_scatter(cntv, [bidx], jnp.ones((_L,), jnp.float32),
                               mask=valid)
        plsc.addupdate_scatter(psv, [bidx], best, mask=valid)

    pltpu.sync_copy(cntv, cnt_hbm.at[wid])
    pltpu.sync_copy(psv, psum_hbm.at[wid])


_sc_call = pl.kernel(
    _sc_body,
    out_type=[
        jax.ShapeDtypeStruct((_NW, _E), jnp.float32),
        jax.ShapeDtypeStruct((_NW, _E), jnp.float32),
    ],
    mesh=plsc.VectorSubcoreMesh(core_axis_name="c", subcore_axis_name="s",
                                num_cores=_NC, num_subcores=_NS),
    compiler_params=pltpu.CompilerParams(needs_layout_passes=False),
    scratch_types=[
        pltpu.VMEM((_TPW * _E,), jnp.float32),
        pltpu.VMEM((_TPW,), jnp.int32),
        pltpu.VMEM((_E,), jnp.float32),
        pltpu.VMEM((_E,), jnp.float32),
    ],
)


def _combine_body(cnt_ref, ps_ref, out_ref):
    cnt = jnp.sum(cnt_ref[...], axis=0, keepdims=True)
    ps = jnp.sum(ps_ref[...], axis=0, keepdims=True)
    nv = jnp.sum(cnt)
    loss = _E * jnp.sum(cnt * ps) / (nv * nv * nv)
    out_ref[...] = jnp.full((1, 1), loss, dtype=jnp.float32)


def kernel(x, mask):
    xr = x.reshape(_N * _E)
    mr = mask.reshape(_N)
    cnt, psum = _sc_call(xr, mr)
    out = pl.pallas_call(
        _combine_body,
        out_shape=jax.ShapeDtypeStruct((1, 1), jnp.float32),
    )(cnt, psum)
    return out[0, 0]


# speedup vs baseline: 1.0165x; 1.0165x over previous
import functools
import jax
import jax.numpy as jnp
from jax import lax
from jax.experimental import pallas as pl
from jax.experimental.pallas import tpu as pltpu
from jax.experimental.pallas import tpu_sc as plsc

_E = 64
_N = 32768
_NC = 2
_NS = 16
_NW = _NC * _NS
_TPW = _N // _NW
_L = 16
_GROUPS = _TPW // _L


def _sc_body(x_hbm, mask_hbm, cnt_hbm, psum_hbm, xv, mv, cntv, psv):
    wid = lax.axis_index("s") * _NC + lax.axis_index("c")
    base = wid * _TPW
    pltpu.sync_copy(x_hbm.at[pl.ds(base * _E, _TPW * _E)], xv)
    pltpu.sync_copy(mask_hbm.at[pl.ds(base, _TPW)], mv)

    zeros = jnp.zeros((_L,), jnp.float32)
    for k in range(_E // _L):
        cntv[pl.ds(_L * k, _L)] = zeros
        psv[pl.ds(_L * k, _L)] = zeros

    lanes = lax.iota(jnp.int32, _L)

    @pl.loop(0, _GROUPS)
    def _grp(g):
        idx0 = g * (_L * _E) + lanes * _E
        valid = mv[pl.ds(g * _L, _L)] == 1

        def pair(va, ia, vb, ib):
            gt = vb > va
            return jnp.where(gt, vb, va), jnp.where(gt, ib, ia)

        winners = []
        for sub in range(_E // 8):
            vals = [plsc.load_gather(xv, [idx0 + (sub * 8 + j)])
                    for j in range(8)]
            lvl = [pair(vals[2 * j],
                        jnp.int32(sub * 8 + 2 * j),
                        vals[2 * j + 1],
                        jnp.int32(sub * 8 + 2 * j + 1))
                   for j in range(4)]
            lvl = [pair(*lvl[2 * j], *lvl[2 * j + 1]) for j in range(2)]
            winners.append(pair(*lvl[0], *lvl[1]))
        while len(winners) > 1:
            winners = [pair(*winners[2 * j], *winners[2 * j + 1])
                       for j in range(len(winners) // 2)]
        best, bidx = winners[0]

        plsc.addupdate_scatter(cntv, [bidx], jnp.ones((_L,), jnp.float32),
                               mask=valid)
        plsc.addupdate_scatter(psv, [bidx], best, mask=valid)

    pltpu.sync_copy(cntv, cnt_hbm.at[wid])
    pltpu.sync_copy(psv, psum_hbm.at[wid])


_sc_call = pl.kernel(
    _sc_body,
    out_type=[
        jax.ShapeDtypeStruct((_NW, _E), jnp.float32),
        jax.ShapeDtypeStruct((_NW, _E), jnp.float32),
    ],
    mesh=plsc.VectorSubcoreMesh(core_axis_name="c", subcore_axis_name="s",
                                num_cores=_NC, num_subcores=_NS),
    compiler_params=pltpu.CompilerParams(needs_layout_passes=False),
    scratch_types=[
        pltpu.VMEM((_TPW * _E,), jnp.float32),
        pltpu.VMEM((_TPW,), jnp.int32),
        pltpu.VMEM((_E,), jnp.float32),
        pltpu.VMEM((_E,), jnp.float32),
    ],
)


def _combine_body(cnt_ref, ps_ref, out_ref):
    cnt = jnp.sum(cnt_ref[...], axis=0, keepdims=True)
    ps = jnp.sum(ps_ref[...], axis=0, keepdims=True)
    nv = jnp.sum(cnt)
    loss = _E * jnp.sum(cnt * ps) / (nv * nv * nv)
    out_ref[...] = jnp.full((1, 1), loss, dtype=jnp.float32)


def kernel(x, mask):
    xr = x.reshape(_N * _E)
    mr = mask.reshape(_N)
    cnt, psum = _sc_call(xr, mr)
    out = pl.pallas_call(
        _combine_body,
        out_shape=jax.ShapeDtypeStruct((1, 1), jnp.float32),
    )(cnt, psum)
    return out[0, 0]


# speedup vs baseline: 1.7281x; 1.7000x over previous
import jax
import jax.numpy as jnp
from jax import lax
from jax.experimental import pallas as pl
from jax.experimental.pallas import tpu as pltpu
from jax.experimental.pallas import tpu_sc as plsc

_E = 64
_N = 32768
_NC = 2
_NS = 16
_NW = _NC * _NS
_TPW = _N // _NW
_L = 16
_CHUNK = 512
_NCHUNK = _TPW // _CHUNK
_GPC = _CHUNK // _L


def _sc_body(x_hbm, mask_hbm, cnt_hbm, psum_hbm, xv, mv, cntv, psv):
    wid = lax.axis_index("s") * _NC + lax.axis_index("c")
    base = wid * _TPW
    pltpu.sync_copy(mask_hbm.at[pl.ds(base, _TPW)], mv)

    zeros = jnp.zeros((_L,), jnp.float32)
    for k in range(_E // _L):
        cntv[pl.ds(_L * k, _L)] = zeros
        psv[pl.ds(_L * k, _L)] = zeros

    lanes = lax.iota(jnp.int32, _L)

    def pair(va, ia, vb, ib):
        takeb = (vb > va) | ((vb == va) & (ib < ia))
        return jnp.where(takeb, vb, va), jnp.where(takeb, ib, ia)

    for c in range(_NCHUNK):
        pltpu.sync_copy(x_hbm.at[pl.ds(base + c * _CHUNK, _CHUNK)], xv)

        @pl.loop(0, _GPC)
        def _grp(g):
            row = g * _L + lanes
            valid = mv[pl.ds(c * _CHUNK + g * _L, _L)] == 1
            ents = []
            for s in range(_E):
                ev = (lanes + s) & (_E - 1)
                ents.append((plsc.load_gather(xv, [row, ev]), ev))
            while len(ents) > 1:
                ents = [pair(*ents[2 * j], *ents[2 * j + 1])
                        for j in range(len(ents) // 2)]
            best, bidx = ents[0]
            plsc.addupdate_scatter(cntv, [bidx], jnp.ones((_L,), jnp.float32),
                                   mask=valid)
            plsc.addupdate_scatter(psv, [bidx], best, mask=valid)

    pltpu.sync_copy(cntv, cnt_hbm.at[wid])
    pltpu.sync_copy(psv, psum_hbm.at[wid])


_sc_call = pl.kernel(
    _sc_body,
    out_type=[
        jax.ShapeDtypeStruct((_NW, _E), jnp.float32),
        jax.ShapeDtypeStruct((_NW, _E), jnp.float32),
    ],
    mesh=plsc.VectorSubcoreMesh(core_axis_name="c", subcore_axis_name="s",
                                num_cores=_NC, num_subcores=_NS),
    compiler_params=pltpu.CompilerParams(needs_layout_passes=False),
    scratch_types=[
        pltpu.VMEM((_CHUNK, _E), jnp.float32),
        pltpu.VMEM((_TPW,), jnp.int32),
        pltpu.VMEM((_E,), jnp.float32),
        pltpu.VMEM((_E,), jnp.float32),
    ],
)


def _combine_body(cnt_ref, ps_ref, out_ref):
    cnt = jnp.sum(cnt_ref[...], axis=0, keepdims=True)
    ps = jnp.sum(ps_ref[...], axis=0, keepdims=True)
    nv = jnp.sum(cnt)
    loss = _E * jnp.sum(cnt * ps) / (nv * nv * nv)
    out_ref[...] = jnp.full((1, 1), loss, dtype=jnp.float32)


def kernel(x, mask):
    xr = x.reshape(_N, _E)
    mr = mask.reshape(_N)
    cnt, psum = _sc_call(xr, mr)
    out = pl.pallas_call(
        _combine_body,
        out_shape=jax.ShapeDtypeStruct((1, 1), jnp.float32),
    )(cnt, psum)
    return out[0, 0]


# speedup vs baseline: 1.7956x; 1.0391x over previous
import jax
import jax.numpy as jnp
from jax import lax
from jax.experimental import pallas as pl
from jax.experimental.pallas import tpu as pltpu
from jax.experimental.pallas import tpu_sc as plsc

_E = 64
_N = 32768
_NC = 2
_NS = 16
_NW = _NC * _NS
_TPW = _N // _NW
_L = 16
_CHUNK = 512
_NCHUNK = _TPW // _CHUNK
_GPC = _CHUNK // _L


def _sc_body(x_hbm, mask_hbm, cnt_hbm, psum_hbm, xv, mv, cntv, psv):
    wid = lax.axis_index("s") * _NC + lax.axis_index("c")
    base = wid * _TPW
    pltpu.sync_copy(mask_hbm.at[pl.ds(base, _TPW)], mv)

    zeros = jnp.zeros((_L,), jnp.float32)
    for k in range(_E // _L):
        cntv[pl.ds(_L * k, _L)] = zeros
        psv[pl.ds(_L * k, _L)] = zeros

    lanes = lax.iota(jnp.int32, _L)

    def pair(va, ia, vb, ib):
        takeb = (vb > va) | ((vb == va) & (ib < ia))
        return jnp.where(takeb, vb, va), jnp.where(takeb, ib, ia)

    for c in range(_NCHUNK):
        pltpu.sync_copy(x_hbm.at[pl.ds(base + c * _CHUNK, _CHUNK)], xv)

        @pl.loop(0, _GPC)
        def _grp(g):
            row = g * _L + lanes
            valid = mv[pl.ds(c * _CHUNK + g * _L, _L)] == 1
            winners = []
            for sub in range(_E // 8):
                ents = []
                for j in range(8):
                    s = sub * 8 + j
                    ev = (lanes + s) & (_E - 1)
                    ents.append((plsc.load_gather(xv, [row, ev]), ev))
                while len(ents) > 1:
                    ents = [pair(*ents[2 * k], *ents[2 * k + 1])
                            for k in range(len(ents) // 2)]
                winners.append(ents[0])
            while len(winners) > 1:
                winners = [pair(*winners[2 * k], *winners[2 * k + 1])
                           for k in range(len(winners) // 2)]
            best, bidx = winners[0]
            plsc.addupdate_scatter(cntv, [bidx], jnp.ones((_L,), jnp.float32),
                                   mask=valid)
            plsc.addupdate_scatter(psv, [bidx], best, mask=valid)

    pltpu.sync_copy(cntv, cnt_hbm.at[wid])
    pltpu.sync_copy(psv, psum_hbm.at[wid])


_sc_call = pl.kernel(
    _sc_body,
    out_type=[
        jax.ShapeDtypeStruct((_NW, _E), jnp.float32),
        jax.ShapeDtypeStruct((_NW, _E), jnp.float32),
    ],
    mesh=plsc.VectorSubcoreMesh(core_axis_name="c", subcore_axis_name="s",
                                num_cores=_NC, num_subcores=_NS),
    compiler_params=pltpu.CompilerParams(needs_layout_passes=False),
    scratch_types=[
        pltpu.VMEM((_CHUNK, _E), jnp.float32),
        pltpu.VMEM((_TPW,), jnp.int32),
        pltpu.VMEM((_E,), jnp.float32),
        pltpu.VMEM((_E,), jnp.float32),
    ],
)


def _combine_body(cnt_ref, ps_ref, out_ref):
    cnt = jnp.sum(cnt_ref[...], axis=0, keepdims=True)
    ps = jnp.sum(ps_ref[...], axis=0, keepdims=True)
    nv = jnp.sum(cnt)
    loss = _E * jnp.sum(cnt * ps) / (nv * nv * nv)
    out_ref[...] = jnp.full((1, 1), loss, dtype=jnp.float32)


def kernel(x, mask):
    xr = x.reshape(_N, _E)
    mr = mask.reshape(_N)
    cnt, psum = _sc_call(xr, mr)
    out = pl.pallas_call(
        _combine_body,
        out_shape=jax.ShapeDtypeStruct((1, 1), jnp.float32),
    )(cnt, psum)
    return out[0, 0]
